# baseline (speedup 1.0000x reference)
import functools
import math

import jax
import jax.numpy as jnp
from jax import lax
from jax.experimental import pallas as pl
from jax.experimental.pallas import tpu as pltpu
from jax.experimental.pallas import tpu_sc as plsc

DIM = 32
SCALE = math.sqrt(float(DIM))

NC, NS = 2, 16
NW = NC * NS
NX, NY = 4096, 200
XR_PER_W = NX // NW
CR = 4
CB = CR * NY
CHUNKS = XR_PER_W // CR
NBUF = 4
LEAD = 2

_mesh = plsc.VectorSubcoreMesh(core_axis_name="c", subcore_axis_name="s")


@functools.partial(
    pl.kernel,
    out_type=jax.ShapeDtypeStruct((NX, NY, DIM), jnp.float32),
    mesh=_mesh,
    compiler_params=pltpu.CompilerParams(use_tc_tiling_on_sc=False),
    scratch_types=[
        pltpu.VMEM((NBUF, CR, NY), jnp.int32),
        pltpu.VMEM((NBUF, CR, NY, DIM), jnp.float32),
    ]
    + [pltpu.SemaphoreType.DMA] * NBUF
    + [pltpu.SemaphoreType.DMA] * NBUF,
)
def _embed_sc(x_hbm, table_hbm, out_hbm, idx_v, rows_v, *sems):
    gsems, osems = sems[:NBUF], sems[NBUF:]
    wid = lax.axis_index("s") * NC + lax.axis_index("c")
    xr0 = wid * XR_PER_W

    def fire_gather(ci, slot):
        pltpu.sync_copy(x_hbm.at[pl.ds(xr0 + ci * CR, CR)], idx_v.at[slot])
        for r in range(CR):
            pltpu.async_copy(
                table_hbm.at[idx_v.at[slot].at[r, pl.ds(0, 128)]],
                rows_v.at[slot].at[r].at[pl.ds(0, 128)],
                gsems[slot],
            )
            pltpu.async_copy(
                table_hbm.at[idx_v.at[slot].at[r, pl.ds(128, NY - 128)]],
                rows_v.at[slot].at[r].at[pl.ds(128, NY - 128)],
                gsems[slot],
            )

    def wait_gather(slot):
        pltpu.make_async_copy(
            out_hbm.at[pl.ds(0, CR)], rows_v.at[slot], gsems[slot]
        ).wait()

    def fire_wb(ci, slot):
        pltpu.async_copy(
            rows_v.at[slot], out_hbm.at[pl.ds(xr0 + ci * CR, CR)], osems[slot]
        )

    def wait_wb(slot):
        pltpu.make_async_copy(
            rows_v.at[slot], out_hbm.at[pl.ds(xr0, CR)], osems[slot]
        ).wait()

    for ci in range(LEAD):
        fire_gather(ci, ci)

    def outer(g, carry):
        for b in range(NBUF):
            ci = g * NBUF + b
            wait_gather(b)

            fslot = (b + LEAD) % NBUF
            cn = ci + LEAD

            @pl.when(jnp.logical_and(cn < CHUNKS, ci >= NBUF - LEAD))
            def _():
                wait_wb(fslot)

            @pl.when(cn < CHUNKS)
            def _():
                fire_gather(cn, fslot)

            for r in range(CR):
                rv = rows_v.at[b].at[r]

                @plsc.parallel_loop(0, NY, 1, unroll=8)
                def _(i):
                    rv[i, pl.ds(0, 16)] = rv[i, pl.ds(0, 16)] * SCALE
                    rv[i, pl.ds(16, 16)] = rv[i, pl.ds(16, 16)] * SCALE

            fire_wb(ci, b)
        return carry

    lax.fori_loop(0, CHUNKS // NBUF, outer, 0)

    for slot in range(NBUF):
        wait_wb(slot)


def kernel(x, table):
    return _embed_sc(x.astype(jnp.int32), table)


# speedup vs baseline: 1.1070x; 1.1070x over previous
import functools
import math

import jax
import jax.numpy as jnp
from jax import lax
from jax.experimental import pallas as pl
from jax.experimental.pallas import tpu as pltpu
from jax.experimental.pallas import tpu_sc as plsc

DIM = 32
SCALE = math.sqrt(float(DIM))

NC, NS = 2, 16
NW = NC * NS
NA, NP = 4096, 200
AB = NA // NW
PG = NP // 8
NBUF = 4
LEAD = 2

_mesh = plsc.VectorSubcoreMesh(core_axis_name="c", subcore_axis_name="s")


@functools.partial(
    pl.kernel,
    out_type=jax.ShapeDtypeStruct((NP, DIM // 8, NW, 8, 128), jnp.float32),
    mesh=_mesh,
    compiler_params=pltpu.CompilerParams(
        use_tc_tiling_on_sc=False, needs_layout_passes=False
    ),
    scratch_types=[
        pltpu.VMEM((PG, 8, 128), jnp.int32),
        pltpu.VMEM((NBUF, 128, DIM), jnp.float32),
        pltpu.VMEM((NBUF, DIM // 8, 8, 128), jnp.float32),
    ]
    + [pltpu.SemaphoreType.DMA] * NBUF
    + [pltpu.SemaphoreType.DMA] * NBUF,
)
def _embed_sc(x_hbm, table_hbm, out_hbm, x_v, rows_v, t_v, *sems):
    gsems, osems = sems[:NBUF], sems[NBUF:]
    wid = lax.axis_index("s") * NC + lax.axis_index("c")

    pltpu.sync_copy(x_hbm.at[:, wid], x_v)

    l_base = lax.iota(jnp.int32, 16)

    def fire_gather(p, slot):
        pg = p // 8
        sx = lax.rem(p, 8)
        pltpu.async_copy(
            table_hbm.at[x_v.at[pg, sx]], rows_v.at[slot], gsems[slot]
        )

    def wait_gather(slot):
        pltpu.make_async_copy(
            table_hbm.at[pl.ds(0, 128)], rows_v.at[slot], gsems[slot]
        ).wait()

    def fire_wb(p, slot):
        pltpu.async_copy(t_v.at[slot], out_hbm.at[p, :, wid], osems[slot])

    def wait_wb(slot):
        pltpu.make_async_copy(
            t_v.at[slot], out_hbm.at[0, :, 0], osems[slot]
        ).wait()

    for p in range(LEAD):
        fire_gather(p, p)

    def outer(g, carry):
        for b in range(NBUF):
            p = g * NBUF + b
            wait_gather(b)

            fslot = (b + LEAD) % NBUF
            pn = p + LEAD

            @pl.when(jnp.logical_and(pn < NP, p >= NBUF - LEAD))
            def _():
                wait_wb(fslot)

            @pl.when(pn < NP)
            def _():
                fire_gather(pn, fslot)

            rows = rows_v.at[b]
            for r in range(DIM // 8):
                tr = t_v.at[b].at[r]

                @plsc.parallel_loop(0, 8, 1, unroll=8)
                def _(sd):
                    d_idx = jnp.zeros((16,), jnp.int32) + (r * 8 + sd)
                    for m in range(8):
                        vals = plsc.load_gather(
                            rows, [l_base + (m * 16), d_idx]
                        )
                        tr[sd, pl.ds(m * 16, 16)] = vals * SCALE

            fire_wb(p, b)
        return carry

    lax.fori_loop(0, NP // NBUF, outer, 0)

    for slot in range(NBUF):
        wait_wb(slot)


def kernel(x, table):
    xt = (
        x.astype(jnp.int32)
        .T.reshape(PG, 8, NW, 128)
        .transpose(0, 2, 1, 3)
    )
    v = _embed_sc(xt, table)
    return v.transpose(2, 4, 0, 1, 3).reshape(NA, NP, DIM)
